# baseline (speedup 1.0000x reference)
import jax
import jax.numpy as jnp
from jax import lax
from jax.experimental import pallas as pl
from jax.experimental.pallas import tpu as pltpu
from jax.experimental.pallas import tpu_sc as plsc

EMBED_DIM = 1024
T_LEN = 8192
N_TILES = 32
LANES = 16
CHUNK = 16
SC_ROWS = 2048
N_CHUNKS = SC_ROWS // (N_TILES * CHUNK)
NBUF = 4
GROUPS = EMBED_DIM // LANES
TC_BLK = 512


def _add_window(dst, src_fn, n_rows, upg=16):
    per_row = GROUPS // upg
    def body(i, carry):
        r = lax.div(i, per_row)
        jb = lax.rem(i, per_row)
        for u in range(upg):
            sl = pl.ds(jb * (upg * LANES) + u * LANES, LANES)
            plsc.addupdate(dst.at[r, sl], src_fn(r, sl))
        return carry
    lax.fori_loop(0, n_rows * per_row, body, 0)


def _sc_body(bar_ref, qn_ref, bar8_ref, glob_ref, out_ref,
             cwin, g0, g1, g2, g3, barbuf, qnbuf,
             ls0, ls1, ls2, ls3, ss0, ss1, ss2, ss3):
    cid = lax.axis_index("c")
    sid = lax.axis_index("s")
    wid = cid * 16 + sid
    gbuf = (g0, g1, g2, g3)
    lsem = (ls0, ls1, ls2, ls3)
    ssem = (ss0, ss1, ss2, ss3)

    def base(k):
        return wid * CHUNK + k * (N_TILES * CHUNK)

    loads = {}
    stores = {}
    for k in range(min(NBUF, N_CHUNKS)):
        loads[k] = pltpu.async_copy(
            glob_ref.at[pl.ds(base(k), CHUNK)], gbuf[k], lsem[k])

    win = (wid % 8) * CHUNK
    pltpu.sync_copy(bar8_ref.at[pl.ds(win, CHUNK)], cwin)
    pltpu.sync_copy(bar_ref, barbuf)
    pltpu.sync_copy(qn_ref, qnbuf)
    _add_window(cwin, lambda r, sl: barbuf[r, sl] + qnbuf[r % 4, sl], CHUNK)

    for k in range(N_CHUNKS):
        b = k % NBUF
        loads[k].wait()
        _add_window(gbuf[b], lambda r, sl: cwin[r, sl], CHUNK)
        stores[k] = pltpu.async_copy(
            gbuf[b], out_ref.at[pl.ds(base(k), CHUNK)], ssem[b])
        lc = k + 2
        if NBUF <= lc < N_CHUNKS:
            stores[lc - NBUF].wait()
            loads[lc] = pltpu.async_copy(
                glob_ref.at[pl.ds(base(lc), CHUNK)], gbuf[lc % NBUF],
                lsem[lc % NBUF])
    for k in range(max(0, N_CHUNKS - NBUF), N_CHUNKS):
        stores[k].wait()


def _tc_body(bar_ref, qn_ref, bar8_ref, glob_ref, out_ref):
    c = (bar8_ref[...]
         + jnp.tile(bar_ref[...], (8, 1))
         + jnp.tile(qn_ref[...], (32, 1)))
    out_ref[...] = glob_ref[...] + jnp.tile(c, (TC_BLK // 128, 1))


def kernel(x, bar_w, qn_w, bar8_w, global_w):
    del x

    mesh = plsc.VectorSubcoreMesh(core_axis_name="c", subcore_axis_name="s",
                                  num_cores=2, num_subcores=16)
    sc_fn = pl.kernel(
        _sc_body,
        out_type=jax.ShapeDtypeStruct((SC_ROWS, EMBED_DIM), jnp.float32),
        mesh=mesh,
        scratch_types=[
            pltpu.VMEM((CHUNK, EMBED_DIM), jnp.float32),
            pltpu.VMEM((CHUNK, EMBED_DIM), jnp.float32),
            pltpu.VMEM((CHUNK, EMBED_DIM), jnp.float32),
            pltpu.VMEM((CHUNK, EMBED_DIM), jnp.float32),
            pltpu.VMEM((CHUNK, EMBED_DIM), jnp.float32),
            pltpu.VMEM((16, EMBED_DIM), jnp.float32),
            pltpu.VMEM((4, EMBED_DIM), jnp.float32),
            pltpu.SemaphoreType.DMA,
            pltpu.SemaphoreType.DMA,
            pltpu.SemaphoreType.DMA,
            pltpu.SemaphoreType.DMA,
            pltpu.SemaphoreType.DMA,
            pltpu.SemaphoreType.DMA,
            pltpu.SemaphoreType.DMA,
            pltpu.SemaphoreType.DMA,
        ],
    )
    sc_out = sc_fn(bar_w, qn_w, bar8_w, global_w)

    n_tc_blocks = (T_LEN - SC_ROWS) // TC_BLK
    blk0 = SC_ROWS // TC_BLK
    tc_full = pl.pallas_call(
        _tc_body,
        grid=(n_tc_blocks,),
        in_specs=[
            pl.BlockSpec((16, EMBED_DIM), lambda i: (0, 0)),
            pl.BlockSpec((4, EMBED_DIM), lambda i: (0, 0)),
            pl.BlockSpec((128, EMBED_DIM), lambda i: (0, 0)),
            pl.BlockSpec((TC_BLK, EMBED_DIM), lambda i: (blk0 + i, 0)),
        ],
        out_specs=pl.BlockSpec((TC_BLK, EMBED_DIM), lambda i: (blk0 + i, 0)),
        out_shape=jax.ShapeDtypeStruct((T_LEN, EMBED_DIM), jnp.float32),
    )(bar_w, qn_w, bar8_w, global_w)

    pe = lax.dynamic_update_slice(tc_full, sc_out, (0, 0))
    return pe[None, :, :]


# speedup vs baseline: 1.3981x; 1.3981x over previous
import jax
import jax.numpy as jnp
from jax import lax
from jax.experimental import pallas as pl
from jax.experimental.pallas import tpu as pltpu
from jax.experimental.pallas import tpu_sc as plsc

EMBED_DIM = 1024
T_LEN = 8192
N_TILES = 32
LANES = 16
CHUNK = 16
SC_ROWS = 512
N_CHUNKS = SC_ROWS // (N_TILES * CHUNK)
NBUF = 4
GROUPS = EMBED_DIM // LANES
TC_BLK = 512


def _add_window(dst, src_fn, n_rows, upg=16):
    per_row = GROUPS // upg
    def body(i, carry):
        r = lax.div(i, per_row)
        jb = lax.rem(i, per_row)
        for u in range(upg):
            sl = pl.ds(jb * (upg * LANES) + u * LANES, LANES)
            plsc.addupdate(dst.at[r, sl], src_fn(r, sl))
        return carry
    lax.fori_loop(0, n_rows * per_row, body, 0)


def _sc_body(bar_ref, qn_ref, bar8_ref, glob_ref, out_ref,
             cwin, g0, g1, g2, g3, barbuf, qnbuf,
             ls0, ls1, ls2, ls3, ss0, ss1, ss2, ss3):
    cid = lax.axis_index("c")
    sid = lax.axis_index("s")
    wid = cid * 16 + sid
    gbuf = (g0, g1, g2, g3)
    lsem = (ls0, ls1, ls2, ls3)
    ssem = (ss0, ss1, ss2, ss3)

    def base(k):
        return wid * CHUNK + k * (N_TILES * CHUNK)

    loads = {}
    stores = {}
    for k in range(min(NBUF, N_CHUNKS)):
        loads[k] = pltpu.async_copy(
            glob_ref.at[pl.ds(base(k), CHUNK)], gbuf[k], lsem[k])

    win = (wid % 8) * CHUNK
    pltpu.sync_copy(bar8_ref.at[pl.ds(win, CHUNK)], cwin)
    pltpu.sync_copy(bar_ref, barbuf)
    pltpu.sync_copy(qn_ref, qnbuf)
    _add_window(cwin, lambda r, sl: barbuf[r, sl] + qnbuf[r % 4, sl], CHUNK)

    for k in range(N_CHUNKS):
        b = k % NBUF
        loads[k].wait()
        _add_window(gbuf[b], lambda r, sl: cwin[r, sl], CHUNK)
        stores[k] = pltpu.async_copy(
            gbuf[b], out_ref.at[pl.ds(base(k), CHUNK)], ssem[b])
        lc = k + 2
        if NBUF <= lc < N_CHUNKS:
            stores[lc - NBUF].wait()
            loads[lc] = pltpu.async_copy(
                glob_ref.at[pl.ds(base(lc), CHUNK)], gbuf[lc % NBUF],
                lsem[lc % NBUF])
    for k in range(max(0, N_CHUNKS - NBUF), N_CHUNKS):
        stores[k].wait()


def _tc_body(bar_ref, qn_ref, bar8_ref, glob_ref, out_ref):
    c = (bar8_ref[...]
         + jnp.tile(bar_ref[...], (8, 1))
         + jnp.tile(qn_ref[...], (32, 1)))
    out_ref[...] = glob_ref[...] + jnp.tile(c, (TC_BLK // 128, 1))


def kernel(x, bar_w, qn_w, bar8_w, global_w):
    del x

    mesh = plsc.VectorSubcoreMesh(core_axis_name="c", subcore_axis_name="s",
                                  num_cores=2, num_subcores=16)
    sc_fn = pl.kernel(
        _sc_body,
        out_type=jax.ShapeDtypeStruct((SC_ROWS, EMBED_DIM), jnp.float32),
        mesh=mesh,
        scratch_types=[
            pltpu.VMEM((CHUNK, EMBED_DIM), jnp.float32),
            pltpu.VMEM((CHUNK, EMBED_DIM), jnp.float32),
            pltpu.VMEM((CHUNK, EMBED_DIM), jnp.float32),
            pltpu.VMEM((CHUNK, EMBED_DIM), jnp.float32),
            pltpu.VMEM((CHUNK, EMBED_DIM), jnp.float32),
            pltpu.VMEM((16, EMBED_DIM), jnp.float32),
            pltpu.VMEM((4, EMBED_DIM), jnp.float32),
            pltpu.SemaphoreType.DMA,
            pltpu.SemaphoreType.DMA,
            pltpu.SemaphoreType.DMA,
            pltpu.SemaphoreType.DMA,
            pltpu.SemaphoreType.DMA,
            pltpu.SemaphoreType.DMA,
            pltpu.SemaphoreType.DMA,
            pltpu.SemaphoreType.DMA,
        ],
    )
    sc_out = sc_fn(bar_w, qn_w, bar8_w, global_w)

    n_tc_blocks = (T_LEN - SC_ROWS) // TC_BLK
    blk0 = SC_ROWS // TC_BLK
    tc_full = pl.pallas_call(
        _tc_body,
        grid=(n_tc_blocks,),
        in_specs=[
            pl.BlockSpec((16, EMBED_DIM), lambda i: (0, 0)),
            pl.BlockSpec((4, EMBED_DIM), lambda i: (0, 0)),
            pl.BlockSpec((128, EMBED_DIM), lambda i: (0, 0)),
            pl.BlockSpec((TC_BLK, EMBED_DIM), lambda i: (blk0 + i, 0)),
        ],
        out_specs=pl.BlockSpec((TC_BLK, EMBED_DIM), lambda i: (blk0 + i, 0)),
        out_shape=jax.ShapeDtypeStruct((T_LEN, EMBED_DIM), jnp.float32),
    )(bar_w, qn_w, bar8_w, global_w)

    pe = lax.dynamic_update_slice(tc_full, sc_out, (0, 0))
    return pe[None, :, :]


# speedup vs baseline: 1.4000x; 1.0013x over previous
import jax
import jax.numpy as jnp
from jax import lax
from jax.experimental import pallas as pl
from jax.experimental.pallas import tpu as pltpu
from jax.experimental.pallas import tpu_sc as plsc

EMBED_DIM = 1024
T_LEN = 8192
N_TILES = 32
LANES = 16
CHUNK = 16
SC_ROWS = 512
N_CHUNKS = SC_ROWS // (N_TILES * CHUNK)
NBUF = 4
GROUPS = EMBED_DIM // LANES
TC_BLK = 512


def _add_window(dst, src_fn, n_rows, upg=16):
    per_row = GROUPS // upg
    def body(i, carry):
        r = lax.div(i, per_row)
        jb = lax.rem(i, per_row)
        for u in range(upg):
            sl = pl.ds(jb * (upg * LANES) + u * LANES, LANES)
            plsc.addupdate(dst.at[r, sl], src_fn(r, sl))
        return carry
    lax.fori_loop(0, n_rows * per_row, body, 0)


def _sc_body(bar_ref, qn_ref, bar8_ref, glob_ref, out_ref,
             cwin, g0, g1, g2, g3, barbuf, qnbuf,
             ls0, ls1, ls2, ls3, ss0, ss1, ss2, ss3):
    cid = lax.axis_index("c")
    sid = lax.axis_index("s")
    wid = cid * 16 + sid
    gbuf = (g0, g1, g2, g3)
    lsem = (ls0, ls1, ls2, ls3)
    ssem = (ss0, ss1, ss2, ss3)

    def base(k):
        return wid * CHUNK + k * (N_TILES * CHUNK)

    loads = {}
    stores = {}
    for k in range(min(NBUF, N_CHUNKS)):
        loads[k] = pltpu.async_copy(
            glob_ref.at[pl.ds(base(k), CHUNK)], gbuf[k], lsem[k])

    win = (wid % 8) * CHUNK
    pltpu.sync_copy(bar8_ref.at[pl.ds(win, CHUNK)], cwin)
    pltpu.sync_copy(bar_ref, barbuf)
    pltpu.sync_copy(qn_ref, qnbuf)
    _add_window(cwin, lambda r, sl: barbuf[r, sl] + qnbuf[r % 4, sl], CHUNK)

    for k in range(N_CHUNKS):
        b = k % NBUF
        loads[k].wait()
        _add_window(gbuf[b], lambda r, sl: cwin[r, sl], CHUNK)
        stores[k] = pltpu.async_copy(
            gbuf[b], out_ref.at[pl.ds(base(k), CHUNK)], ssem[b])
        lc = k + 2
        if NBUF <= lc < N_CHUNKS:
            stores[lc - NBUF].wait()
            loads[lc] = pltpu.async_copy(
                glob_ref.at[pl.ds(base(lc), CHUNK)], gbuf[lc % NBUF],
                lsem[lc % NBUF])
    for k in range(max(0, N_CHUNKS - NBUF), N_CHUNKS):
        stores[k].wait()


def _tc_body(bar_ref, qn_ref, bar8_ref, glob_ref, out_ref):
    c = (bar8_ref[...]
         + jnp.tile(bar_ref[...], (8, 1))
         + jnp.tile(qn_ref[...], (32, 1)))
    out_ref[...] = glob_ref[...] + jnp.tile(c, (TC_BLK // 128, 1))


def kernel(x, bar_w, qn_w, bar8_w, global_w):
    del x

    n_tc_blocks = (T_LEN - SC_ROWS) // TC_BLK
    blk0 = SC_ROWS // TC_BLK
    tc_full = pl.pallas_call(
        _tc_body,
        grid=(n_tc_blocks,),
        in_specs=[
            pl.BlockSpec((16, EMBED_DIM), lambda i: (0, 0)),
            pl.BlockSpec((4, EMBED_DIM), lambda i: (0, 0)),
            pl.BlockSpec((128, EMBED_DIM), lambda i: (0, 0)),
            pl.BlockSpec((TC_BLK, EMBED_DIM), lambda i: (blk0 + i, 0)),
        ],
        out_specs=pl.BlockSpec((TC_BLK, EMBED_DIM), lambda i: (blk0 + i, 0)),
        out_shape=jax.ShapeDtypeStruct((T_LEN, EMBED_DIM), jnp.float32),
    )(bar_w, qn_w, bar8_w, global_w)

    mesh = plsc.VectorSubcoreMesh(core_axis_name="c", subcore_axis_name="s",
                                  num_cores=2, num_subcores=16)
    sc_fn = pl.kernel(
        _sc_body,
        out_type=jax.ShapeDtypeStruct((SC_ROWS, EMBED_DIM), jnp.float32),
        mesh=mesh,
        scratch_types=[
            pltpu.VMEM((CHUNK, EMBED_DIM), jnp.float32),
            pltpu.VMEM((CHUNK, EMBED_DIM), jnp.float32),
            pltpu.VMEM((CHUNK, EMBED_DIM), jnp.float32),
            pltpu.VMEM((CHUNK, EMBED_DIM), jnp.float32),
            pltpu.VMEM((CHUNK, EMBED_DIM), jnp.float32),
            pltpu.VMEM((16, EMBED_DIM), jnp.float32),
            pltpu.VMEM((4, EMBED_DIM), jnp.float32),
            pltpu.SemaphoreType.DMA,
            pltpu.SemaphoreType.DMA,
            pltpu.SemaphoreType.DMA,
            pltpu.SemaphoreType.DMA,
            pltpu.SemaphoreType.DMA,
            pltpu.SemaphoreType.DMA,
            pltpu.SemaphoreType.DMA,
            pltpu.SemaphoreType.DMA,
        ],
    )
    sc_out = sc_fn(bar_w, qn_w, bar8_w, global_w)

    pe = lax.dynamic_update_slice(tc_full, sc_out, (0, 0))
    return pe[None, :, :]


# speedup vs baseline: 1.4364x; 1.0260x over previous
import jax
import jax.numpy as jnp
from jax import lax
from jax.experimental import pallas as pl
from jax.experimental.pallas import tpu as pltpu
from jax.experimental.pallas import tpu_sc as plsc

EMBED_DIM = 1024
T_LEN = 8192
N_TILES = 32
LANES = 16
C_ROWS = 128
RPT = C_ROWS // N_TILES
GROUPS = EMBED_DIM // LANES
TC_BLK = 512


def _add_window(dst, src_fn, n_rows, upg=16):
    per_row = GROUPS // upg
    def body(i, carry):
        r = lax.div(i, per_row)
        jb = lax.rem(i, per_row)
        for u in range(upg):
            sl = pl.ds(jb * (upg * LANES) + u * LANES, LANES)
            plsc.addupdate(dst.at[r, sl], src_fn(r, sl))
        return carry
    lax.fori_loop(0, n_rows * per_row, body, 0)


def _sc_body(bar_ref, qn_ref, bar8_ref, c_ref,
             cbuf, barbuf, qnbuf, sem0, sem1, sem2):
    cid = lax.axis_index("c")
    sid = lax.axis_index("s")
    wid = cid * 16 + sid
    row0 = wid * RPT

    d0 = pltpu.async_copy(bar8_ref.at[pl.ds(row0, RPT)], cbuf, sem0)
    d1 = pltpu.async_copy(bar_ref.at[pl.ds((wid % 4) * RPT, RPT)], barbuf,
                          sem1)
    d2 = pltpu.async_copy(qn_ref, qnbuf, sem2)
    d0.wait()
    d1.wait()
    d2.wait()
    _add_window(cbuf, lambda r, sl: barbuf[r, sl] + qnbuf[r, sl], RPT)
    pltpu.sync_copy(cbuf, c_ref.at[pl.ds(row0, RPT)])


def _tc_body(c_ref, glob_ref, out_ref):
    out_ref[...] = glob_ref[...] + jnp.tile(c_ref[...], (TC_BLK // C_ROWS, 1))


def kernel(x, bar_w, qn_w, bar8_w, global_w):
    del x

    mesh = plsc.VectorSubcoreMesh(core_axis_name="c", subcore_axis_name="s",
                                  num_cores=2, num_subcores=16)
    sc_fn = pl.kernel(
        _sc_body,
        out_type=jax.ShapeDtypeStruct((C_ROWS, EMBED_DIM), jnp.float32),
        mesh=mesh,
        scratch_types=[
            pltpu.VMEM((RPT, EMBED_DIM), jnp.float32),
            pltpu.VMEM((RPT, EMBED_DIM), jnp.float32),
            pltpu.VMEM((4, EMBED_DIM), jnp.float32),
            pltpu.SemaphoreType.DMA,
            pltpu.SemaphoreType.DMA,
            pltpu.SemaphoreType.DMA,
        ],
    )
    c = sc_fn(bar_w, qn_w, bar8_w)

    pe = pl.pallas_call(
        _tc_body,
        grid=(T_LEN // TC_BLK,),
        in_specs=[
            pl.BlockSpec((C_ROWS, EMBED_DIM), lambda i: (0, 0)),
            pl.BlockSpec((TC_BLK, EMBED_DIM), lambda i: (i, 0)),
        ],
        out_specs=pl.BlockSpec((TC_BLK, EMBED_DIM), lambda i: (i, 0)),
        out_shape=jax.ShapeDtypeStruct((T_LEN, EMBED_DIM), jnp.float32),
    )(c, global_w)

    return pe[None, :, :]


# speedup vs baseline: 1.5027x; 1.0461x over previous
import jax
import jax.numpy as jnp
from jax import lax
from jax.experimental import pallas as pl
from jax.experimental.pallas import tpu as pltpu
from jax.experimental.pallas import tpu_sc as plsc

EMBED_DIM = 1024
T_LEN = 8192
N_TILES = 32
LANES = 16
C_ROWS = 128
RPT = C_ROWS // N_TILES
GROUPS = EMBED_DIM // LANES
TC_BLK = 1024


def _add_window(dst, src_fn, n_rows, upg=16):
    per_row = GROUPS // upg
    def body(i, carry):
        r = lax.div(i, per_row)
        jb = lax.rem(i, per_row)
        for u in range(upg):
            sl = pl.ds(jb * (upg * LANES) + u * LANES, LANES)
            plsc.addupdate(dst.at[r, sl], src_fn(r, sl))
        return carry
    lax.fori_loop(0, n_rows * per_row, body, 0)


def _sc_body(bar_ref, qn_ref, bar8_ref, c_ref,
             cbuf, barbuf, qnbuf, sem0, sem1, sem2):
    cid = lax.axis_index("c")
    sid = lax.axis_index("s")
    wid = cid * 16 + sid
    row0 = wid * RPT

    d0 = pltpu.async_copy(bar8_ref.at[pl.ds(row0, RPT)], cbuf, sem0)
    d1 = pltpu.async_copy(bar_ref.at[pl.ds((wid % 4) * RPT, RPT)], barbuf,
                          sem1)
    d2 = pltpu.async_copy(qn_ref, qnbuf, sem2)
    d0.wait()
    d1.wait()
    d2.wait()
    _add_window(cbuf, lambda r, sl: barbuf[r, sl] + qnbuf[r, sl], RPT)
    pltpu.sync_copy(cbuf, c_ref.at[pl.ds(row0, RPT)])


def _tc_body(c_ref, glob_ref, out_ref):
    out_ref[...] = glob_ref[...] + jnp.tile(c_ref[...], (TC_BLK // C_ROWS, 1))


def kernel(x, bar_w, qn_w, bar8_w, global_w):
    del x

    mesh = plsc.VectorSubcoreMesh(core_axis_name="c", subcore_axis_name="s",
                                  num_cores=2, num_subcores=16)
    sc_fn = pl.kernel(
        _sc_body,
        out_type=jax.ShapeDtypeStruct((C_ROWS, EMBED_DIM), jnp.float32),
        mesh=mesh,
        scratch_types=[
            pltpu.VMEM((RPT, EMBED_DIM), jnp.float32),
            pltpu.VMEM((RPT, EMBED_DIM), jnp.float32),
            pltpu.VMEM((4, EMBED_DIM), jnp.float32),
            pltpu.SemaphoreType.DMA,
            pltpu.SemaphoreType.DMA,
            pltpu.SemaphoreType.DMA,
        ],
    )
    c = sc_fn(bar_w, qn_w, bar8_w)

    pe = pl.pallas_call(
        _tc_body,
        grid=(T_LEN // TC_BLK,),
        in_specs=[
            pl.BlockSpec((C_ROWS, EMBED_DIM), lambda i: (0, 0)),
            pl.BlockSpec((TC_BLK, EMBED_DIM), lambda i: (i, 0)),
        ],
        out_specs=pl.BlockSpec((TC_BLK, EMBED_DIM), lambda i: (i, 0)),
        out_shape=jax.ShapeDtypeStruct((T_LEN, EMBED_DIM), jnp.float32),
    )(c, global_w)

    return pe[None, :, :]


# speedup vs baseline: 1.5555x; 1.0351x over previous
import jax
import jax.numpy as jnp
from jax import lax
from jax.experimental import pallas as pl
from jax.experimental.pallas import tpu as pltpu
from jax.experimental.pallas import tpu_sc as plsc

EMBED_DIM = 1024
T_LEN = 8192
N_TILES = 32
LANES = 16
C_ROWS = 128
RPT = C_ROWS // N_TILES
GROUPS = EMBED_DIM // LANES
TC_BLK = 2048


def _add_window(dst, src_fn, n_rows, upg=16):
    per_row = GROUPS // upg
    def body(i, carry):
        r = lax.div(i, per_row)
        jb = lax.rem(i, per_row)
        for u in range(upg):
            sl = pl.ds(jb * (upg * LANES) + u * LANES, LANES)
            plsc.addupdate(dst.at[r, sl], src_fn(r, sl))
        return carry
    lax.fori_loop(0, n_rows * per_row, body, 0)


def _sc_body(bar_ref, qn_ref, bar8_ref, c_ref,
             cbuf, barbuf, qnbuf, sem0, sem1, sem2):
    cid = lax.axis_index("c")
    sid = lax.axis_index("s")
    wid = cid * 16 + sid
    row0 = wid * RPT

    d0 = pltpu.async_copy(bar8_ref.at[pl.ds(row0, RPT)], cbuf, sem0)
    d1 = pltpu.async_copy(bar_ref.at[pl.ds((wid % 4) * RPT, RPT)], barbuf,
                          sem1)
    d2 = pltpu.async_copy(qn_ref, qnbuf, sem2)
    d0.wait()
    d1.wait()
    d2.wait()
    _add_window(cbuf, lambda r, sl: barbuf[r, sl] + qnbuf[r, sl], RPT)
    pltpu.sync_copy(cbuf, c_ref.at[pl.ds(row0, RPT)])


def _tc_body(c_ref, glob_ref, out_ref):
    out_ref[...] = glob_ref[...] + jnp.tile(c_ref[...], (TC_BLK // C_ROWS, 1))


def kernel(x, bar_w, qn_w, bar8_w, global_w):
    del x

    mesh = plsc.VectorSubcoreMesh(core_axis_name="c", subcore_axis_name="s",
                                  num_cores=2, num_subcores=16)
    sc_fn = pl.kernel(
        _sc_body,
        out_type=jax.ShapeDtypeStruct((C_ROWS, EMBED_DIM), jnp.float32),
        mesh=mesh,
        scratch_types=[
            pltpu.VMEM((RPT, EMBED_DIM), jnp.float32),
            pltpu.VMEM((RPT, EMBED_DIM), jnp.float32),
            pltpu.VMEM((4, EMBED_DIM), jnp.float32),
            pltpu.SemaphoreType.DMA,
            pltpu.SemaphoreType.DMA,
            pltpu.SemaphoreType.DMA,
        ],
    )
    c = sc_fn(bar_w, qn_w, bar8_w)

    pe = pl.pallas_call(
        _tc_body,
        grid=(T_LEN // TC_BLK,),
        in_specs=[
            pl.BlockSpec((C_ROWS, EMBED_DIM), lambda i: (0, 0)),
            pl.BlockSpec((TC_BLK, EMBED_DIM), lambda i: (i, 0)),
        ],
        out_specs=pl.BlockSpec((TC_BLK, EMBED_DIM), lambda i: (i, 0)),
        out_shape=jax.ShapeDtypeStruct((T_LEN, EMBED_DIM), jnp.float32),
    )(c, global_w)

    return pe[None, :, :]


# speedup vs baseline: 1.5574x; 1.0012x over previous
import jax
import jax.numpy as jnp
from jax import lax
from jax.experimental import pallas as pl
from jax.experimental.pallas import tpu as pltpu
from jax.experimental.pallas import tpu_sc as plsc

EMBED_DIM = 1024
T_LEN = 8192
N_TILES = 32
LANES = 16
C_ROWS = 128
RPT = C_ROWS // N_TILES
GROUPS = EMBED_DIM // LANES
TC_BLK = 2048


def _add_window(dst, src_fn, n_rows, upg=16):
    per_row = GROUPS // upg
    def body(i, carry):
        r = lax.div(i, per_row)
        jb = lax.rem(i, per_row)
        for u in range(upg):
            sl = pl.ds(jb * (upg * LANES) + u * LANES, LANES)
            plsc.addupdate(dst.at[r, sl], src_fn(r, sl))
        return carry
    lax.fori_loop(0, n_rows * per_row, body, 0)


def _sc_body(bar_ref, qn_ref, bar8_ref, c_ref,
             cbuf, barbuf, qnbuf, sem0, sem1, sem2):
    cid = lax.axis_index("c")
    sid = lax.axis_index("s")
    wid = cid * 16 + sid
    row0 = wid * RPT

    d0 = pltpu.async_copy(bar8_ref.at[pl.ds(row0, RPT)], cbuf, sem0)
    d1 = pltpu.async_copy(bar_ref.at[pl.ds((wid % 4) * RPT, RPT)], barbuf,
                          sem1)
    d2 = pltpu.async_copy(qn_ref, qnbuf, sem2)
    d0.wait()
    d1.wait()
    d2.wait()
    _add_window(cbuf, lambda r, sl: barbuf[r, sl] + qnbuf[r, sl], RPT)
    pltpu.sync_copy(cbuf, c_ref.at[pl.ds(row0, RPT)])


def _tc_body(c_ref, glob_ref, out_ref):
    c = c_ref[...]
    for j in range(TC_BLK // C_ROWS):
        sl = pl.ds(j * C_ROWS, C_ROWS)
        out_ref[sl, :] = glob_ref[sl, :] + c


def kernel(x, bar_w, qn_w, bar8_w, global_w):
    del x

    mesh = plsc.VectorSubcoreMesh(core_axis_name="c", subcore_axis_name="s",
                                  num_cores=2, num_subcores=16)
    sc_fn = pl.kernel(
        _sc_body,
        out_type=jax.ShapeDtypeStruct((C_ROWS, EMBED_DIM), jnp.float32),
        mesh=mesh,
        scratch_types=[
            pltpu.VMEM((RPT, EMBED_DIM), jnp.float32),
            pltpu.VMEM((RPT, EMBED_DIM), jnp.float32),
            pltpu.VMEM((4, EMBED_DIM), jnp.float32),
            pltpu.SemaphoreType.DMA,
            pltpu.SemaphoreType.DMA,
            pltpu.SemaphoreType.DMA,
        ],
    )
    c = sc_fn(bar_w, qn_w, bar8_w)

    pe = pl.pallas_call(
        _tc_body,
        grid=(T_LEN // TC_BLK,),
        in_specs=[
            pl.BlockSpec((C_ROWS, EMBED_DIM), lambda i: (0, 0)),
            pl.BlockSpec((TC_BLK, EMBED_DIM), lambda i: (i, 0)),
        ],
        out_specs=pl.BlockSpec((TC_BLK, EMBED_DIM), lambda i: (i, 0)),
        out_shape=jax.ShapeDtypeStruct((T_LEN, EMBED_DIM), jnp.float32),
    )(c, global_w)

    return pe[None, :, :]
